# baseline (speedup 1.0000x reference)
import functools

import jax
import jax.numpy as jnp
from jax import lax
from jax.experimental import pallas as pl
from jax.experimental.pallas import tpu as pltpu
from jax.experimental.pallas import tpu_sc as plsc

_TOP_K = 8
_BM = 1024
_NE = 64
_NT = 16384
_NC = 2
_NS = 16
_NW = _NC * _NS
_TPW = _NT // _NW
_GROUPS = _TPW // 16
_EUNROLL = 4


def _matmul_body(x_ref, w_ref, b_ref, o_ref):
    o_ref[...] = jnp.dot(
        x_ref[...], w_ref[...],
        preferred_element_type=jnp.float32) + b_ref[...]


def _tc_logits(x, W, b):
    n_tokens, d = x.shape
    n_exp = W.shape[1]
    b2 = b.reshape(1, n_exp)
    grid = (n_tokens // _BM,)
    return pl.pallas_call(
        _matmul_body,
        grid=grid,
        in_specs=[
            pl.BlockSpec((_BM, d), lambda i: (i, 0)),
            pl.BlockSpec((d, n_exp), lambda i: (0, 0)),
            pl.BlockSpec((1, n_exp), lambda i: (0, 0)),
        ],
        out_specs=pl.BlockSpec((_BM, n_exp), lambda i: (i, 0)),
        out_shape=jax.ShapeDtypeStruct((n_tokens, n_exp), jnp.float32),
    )(x, W, b2)


@functools.partial(
    pl.kernel,
    out_type=jax.ShapeDtypeStruct((_NT * _NE,), jnp.float32),
    mesh=plsc.VectorSubcoreMesh(core_axis_name="c", subcore_axis_name="s"),
    scratch_types=[
        pltpu.VMEM((_TPW * _NE,), jnp.float32),
        pltpu.VMEM((_TPW * _NE,), jnp.float32),
    ],
    compiler_params=pltpu.CompilerParams(needs_layout_passes=False),
)
def _sc_router(lg_hbm, out_hbm, lg_v, ot_v):
    wid = lax.axis_index("s") * _NC + lax.axis_index("c")
    base = wid * _TPW * _NE
    pltpu.sync_copy(lg_hbm.at[pl.ds(base, _TPW * _NE)], lg_v)

    lane = lax.iota(jnp.int32, 16)

    def token_body(tok, carry):
        off = tok * _NE
        v0 = lg_v[pl.ds(off, 16)]
        v1 = lg_v[pl.ds(off + 16, 16)]
        v2 = lg_v[pl.ds(off + 32, 16)]
        v3 = lg_v[pl.ds(off + 48, 16)]
        s0 = jnp.sort(v0)
        s1 = jnp.sort(v1)
        s2 = jnp.sort(v2)
        s3 = jnp.sort(v3)
        s01 = jnp.sort(jnp.maximum(s0, jnp.flip(s1)))
        s23 = jnp.sort(jnp.maximum(s2, jnp.flip(s3)))
        c = jnp.sort(jnp.maximum(s01, jnp.flip(s23)))
        t = jnp.sum(jnp.where(lane == 8, c, jnp.float32(0.0)))
        m = jnp.sum(jnp.where(lane == 15, c, jnp.float32(0.0)))
        t16 = jnp.full((16,), t, jnp.float32)
        m16 = jnp.full((16,), m, jnp.float32)
        e0 = jnp.where(v0 >= t16, jnp.exp(v0 - m16), jnp.float32(0.0))
        e1 = jnp.where(v1 >= t16, jnp.exp(v1 - m16), jnp.float32(0.0))
        e2 = jnp.where(v2 >= t16, jnp.exp(v2 - m16), jnp.float32(0.0))
        e3 = jnp.where(v3 >= t16, jnp.exp(v3 - m16), jnp.float32(0.0))
        denom = jnp.sum(e0 + e1 + e2 + e3)
        d16 = jnp.full((16,), denom, jnp.float32)
        inv16 = jnp.full((16,), jnp.float32(1.0), jnp.float32) / d16
        ot_v[pl.ds(off, 16)] = e0 * inv16
        ot_v[pl.ds(off + 16, 16)] = e1 * inv16
        ot_v[pl.ds(off + 32, 16)] = e2 * inv16
        ot_v[pl.ds(off + 48, 16)] = e3 * inv16
        return carry

    lax.fori_loop(0, _TPW, token_body, 0)
    pltpu.sync_copy(ot_v, out_hbm.at[pl.ds(base, _TPW * _NE)])


def kernel(x, W, b):
    logits = _tc_logits(x, W, b)
    return _sc_router(logits.reshape(_NT * _NE)).reshape(_NT, _NE)


# speedup vs baseline: 1.3923x; 1.3923x over previous
import jax
import jax.numpy as jnp
from jax.experimental import pallas as pl
from jax.experimental.pallas import tpu as pltpu

_TOP_K = 8
_BM = 1024
_NE = 64


def _xor_partner_rows(x, j):
    if j >= 8:
        n = x.shape[0]
        parts = [x[(b ^ 1) * j:((b ^ 1) * j) + j] for b in range(n // j)]
        return jnp.concatenate(parts, axis=0)
    row = jax.lax.broadcasted_iota(jnp.int32, x.shape, dimension=0)
    lower = (row & j) == 0
    return jnp.where(lower, pltpu.roll(x, x.shape[0] - j, 0), pltpu.roll(x, j, 0))


def _bitonic_desc_rows(x):
    n = x.shape[0]
    row = jax.lax.broadcasted_iota(jnp.int32, x.shape, dimension=0)
    for k_sz in (2, 4, 8, 16, 32, 64):
        j = k_sz // 2
        while j >= 1:
            lower = (row & j) == 0
            partner = _xor_partner_rows(x, j)
            mx = jnp.maximum(x, partner)
            mn = jnp.minimum(x, partner)
            if k_sz < n:
                desc = (row & k_sz) == 0
                take_max = jnp.logical_not(jnp.logical_xor(lower, desc))
            else:
                take_max = lower
            x = jnp.where(take_max, mx, mn)
            j //= 2
    return x


def _gating_body(x_ref, w_ref, b_ref, o_ref):
    logits = jnp.dot(x_ref[...], w_ref[...], preferred_element_type=jnp.float32)
    lt = jnp.transpose(logits)
    lt = lt + b_ref[...]
    s = _bitonic_desc_rows(lt)
    t = jnp.broadcast_to(s[_TOP_K - 1:_TOP_K, :], lt.shape)
    m = jnp.broadcast_to(s[0:1, :], lt.shape)
    e = jnp.where(lt >= t, jnp.exp(lt - m), 0.0)
    d = e[0:32] + e[32:64]
    d = d[0:16] + d[16:32]
    d = d[0:8] + d[8:16]
    d = d + pltpu.roll(d, 4, 0)
    d = d + pltpu.roll(d, 2, 0)
    d = d + pltpu.roll(d, 1, 0)
    inv = 1.0 / d
    ot = e * jnp.concatenate([inv] * 8, axis=0)
    o_ref[...] = jnp.transpose(ot)


def kernel(x, W, b):
    n_tokens, d = x.shape
    n_exp = W.shape[1]
    b2 = b.reshape(n_exp, 1)
    grid = (n_tokens // _BM,)
    return pl.pallas_call(
        _gating_body,
        grid=grid,
        in_specs=[
            pl.BlockSpec((_BM, d), lambda i: (i, 0)),
            pl.BlockSpec((d, n_exp), lambda i: (0, 0)),
            pl.BlockSpec((n_exp, 1), lambda i: (0, 0)),
        ],
        out_specs=pl.BlockSpec((_BM, n_exp), lambda i: (i, 0)),
        out_shape=jax.ShapeDtypeStruct((n_tokens, n_exp), jnp.float32),
    )(x, W, b2)


# speedup vs baseline: 1.5505x; 1.1136x over previous
import jax
import jax.numpy as jnp
from jax.experimental import pallas as pl
from jax.experimental.pallas import tpu as pltpu

_TOP_K = 8
_BM = 1024
_NE = 64


def _xor_partner_rows(x, j):
    if j >= 8:
        n = x.shape[0]
        parts = [x[(b ^ 1) * j:((b ^ 1) * j) + j] for b in range(n // j)]
        return jnp.concatenate(parts, axis=0)
    row = jax.lax.broadcasted_iota(jnp.int32, x.shape, dimension=0)
    lower = (row & j) == 0
    return jnp.where(lower, pltpu.roll(x, x.shape[0] - j, 0), pltpu.roll(x, j, 0))


def _bitonic_desc_rows(x):
    n = x.shape[0]
    row = jax.lax.broadcasted_iota(jnp.int32, x.shape, dimension=0)
    for k_sz in (2, 4, 8, 16, 32, 64):
        j = k_sz // 2
        while j >= 1:
            lower = (row & j) == 0
            partner = _xor_partner_rows(x, j)
            mx = jnp.maximum(x, partner)
            mn = jnp.minimum(x, partner)
            if k_sz < n:
                desc = (row & k_sz) == 0
                take_max = jnp.logical_not(jnp.logical_xor(lower, desc))
            else:
                take_max = lower
            x = jnp.where(take_max, mx, mn)
            j //= 2
    return x


def _gating_body(x_ref, w_ref, b_ref, o_ref):
    logits = jnp.dot(x_ref[...], w_ref[...], preferred_element_type=jnp.float32)
    logits = logits + b_ref[...]
    lt = jnp.transpose(logits)
    s = _bitonic_desc_rows(lt)
    t = jnp.broadcast_to(s[_TOP_K - 1:_TOP_K, :], lt.shape)
    m = jnp.broadcast_to(s[0:1, :], lt.shape)
    e = jnp.where(lt >= t, jnp.exp(lt - m), 0.0)
    d = e[0:32] + e[32:64]
    d = d[0:16] + d[16:32]
    d = d[0:8] + d[8:16]
    d = d + pltpu.roll(d, 4, 0)
    d = d + pltpu.roll(d, 2, 0)
    d = d + pltpu.roll(d, 1, 0)
    inv = 1.0 / d
    o_ref[...] = e * jnp.concatenate([inv] * 8, axis=0)


def kernel(x, W, b):
    n_tokens, d = x.shape
    n_exp = W.shape[1]
    b2 = b.reshape(1, n_exp)
    grid = (n_tokens // _BM,)
    out_t = pl.pallas_call(
        _gating_body,
        grid=grid,
        in_specs=[
            pl.BlockSpec((_BM, d), lambda i: (i, 0)),
            pl.BlockSpec((d, n_exp), lambda i: (0, 0)),
            pl.BlockSpec((1, n_exp), lambda i: (0, 0)),
        ],
        out_specs=pl.BlockSpec((n_exp, _BM), lambda i: (0, i)),
        out_shape=jax.ShapeDtypeStruct((n_exp, n_tokens), jnp.float32),
    )(x, W, b2)
    return jnp.transpose(out_t)


# speedup vs baseline: 1.6112x; 1.0391x over previous
import jax
import jax.numpy as jnp
from jax.experimental import pallas as pl
from jax.experimental.pallas import tpu as pltpu

_TOP_K = 8
_BM = 1024
_NE = 64


def _xor_partner_rows(x, j):
    if j >= 8:
        n = x.shape[0]
        parts = [x[(b ^ 1) * j:((b ^ 1) * j) + j] for b in range(n // j)]
        return jnp.concatenate(parts, axis=0)
    row = jax.lax.broadcasted_iota(jnp.int32, x.shape, dimension=0)
    lower = (row & j) == 0
    return jnp.where(lower, pltpu.roll(x, x.shape[0] - j, 0), pltpu.roll(x, j, 0))


def _bitonic_desc_rows(x):
    n = x.shape[0]
    row = jax.lax.broadcasted_iota(jnp.int32, x.shape, dimension=0)
    for k_sz in (2, 4, 8, 16, 32, 64):
        j = k_sz // 2
        while j >= 1:
            lower = (row & j) == 0
            partner = _xor_partner_rows(x, j)
            mx = jnp.maximum(x, partner)
            mn = jnp.minimum(x, partner)
            if k_sz < n:
                desc = (row & k_sz) == 0
                take_max = jnp.logical_not(jnp.logical_xor(lower, desc))
            else:
                take_max = lower
            x = jnp.where(take_max, mx, mn)
            j //= 2
    return x


def _gating_body(x_ref, w_ref, b_ref, o_ref):
    logits = jax.lax.dot_general(
        x_ref[...], w_ref[...], (((1,), (1,)), ((), ())),
        preferred_element_type=jnp.float32)
    logits = logits + b_ref[...]
    lt = jnp.transpose(logits)
    s = _bitonic_desc_rows(lt)
    t = jnp.broadcast_to(s[_TOP_K - 1:_TOP_K, :], lt.shape)
    m = jnp.broadcast_to(s[0:1, :], lt.shape)
    e = jnp.where(lt >= t, jnp.exp(lt - m), 0.0)
    d = e[0:32] + e[32:64]
    d = d[0:16] + d[16:32]
    d = d[0:8] + d[8:16]
    d = d + pltpu.roll(d, 4, 0)
    d = d + pltpu.roll(d, 2, 0)
    d = d + pltpu.roll(d, 1, 0)
    inv = 1.0 / d
    o_ref[...] = e * jnp.concatenate([inv] * 8, axis=0)


def kernel(x, W, b):
    n_tokens, d = x.shape
    n_exp = W.shape[1]
    b2 = b.reshape(1, n_exp)
    grid = (n_tokens // _BM,)
    out_t = pl.pallas_call(
        _gating_body,
        grid=grid,
        in_specs=[
            pl.BlockSpec((_BM, d), lambda i: (i, 0)),
            pl.BlockSpec((n_exp, d), lambda i: (0, 0)),
            pl.BlockSpec((1, n_exp), lambda i: (0, 0)),
        ],
        out_specs=pl.BlockSpec((n_exp, _BM), lambda i: (0, i)),
        out_shape=jax.ShapeDtypeStruct((n_exp, n_tokens), jnp.float32),
    )(x, jnp.transpose(W), b2)
    return jnp.transpose(out_t)
